# baseline (speedup 1.0000x reference)
import jax
import jax.numpy as jnp
from jax import lax
from jax.experimental import pallas as pl
from jax.experimental.pallas import tpu as pltpu
from jax.experimental.pallas import tpu_sc as plsc

N = 320000
C = 128
CQ = 32
NSEG = 10000
S_PAD = 10112
EPS_BN = 1e-3
NC = 2
NS = 16
K1B = 2000
K2K = 80
K4K = 80
K3B = 1264


def _k1_body(x_ref, xyz_ref, w1t_ref, b1_ref, wpt_ref, bp_ref,
             y_ref, pw_ref, stats_ref, acc_ref):
    i = pl.program_id(0)
    y = jnp.dot(x_ref[...], w1t_ref[...],
                preferred_element_type=jnp.float32) + b1_ref[...]
    p = jnp.floor(xyz_ref[...])
    wpt = wpt_ref[...]
    pw = (p[:, 0:1] * wpt[0:1, :] + p[:, 1:2] * wpt[1:2, :]
          + p[:, 2:3] * wpt[2:3, :] + bp_ref[...])
    for q in range(4):
        y_ref[q] = y[:, q * CQ:(q + 1) * CQ]
        pw_ref[q] = pw[:, q * CQ:(q + 1) * CQ]

    @pl.when(i == 0)
    def _():
        acc_ref[...] = jnp.zeros_like(acc_ref)

    acc_ref[0:1, :] = acc_ref[0:1, :] + jnp.sum(y, axis=0, keepdims=True)
    acc_ref[1:2, :] = acc_ref[1:2, :] + jnp.sum(y * y, axis=0, keepdims=True)

    @pl.when(i == pl.num_programs(0) - 1)
    def _():
        stats_ref[...] = acc_ref[...]


def _k1(feat_all, xyz, w1t, b1r, wpt, bpr):
    nb = N // K1B
    return pl.pallas_call(
        _k1_body,
        grid=(nb,),
        in_specs=[
            pl.BlockSpec((K1B, C), lambda i: (i, 0)),
            pl.BlockSpec((K1B, 3), lambda i: (i, 0)),
            pl.BlockSpec((C, C), lambda i: (0, 0)),
            pl.BlockSpec((1, C), lambda i: (0, 0)),
            pl.BlockSpec((3, C), lambda i: (0, 0)),
            pl.BlockSpec((1, C), lambda i: (0, 0)),
        ],
        out_specs=[
            pl.BlockSpec((4, K1B, CQ), lambda i: (0, i, 0)),
            pl.BlockSpec((4, K1B, CQ), lambda i: (0, i, 0)),
            pl.BlockSpec((2, C), lambda i: (0, 0)),
        ],
        out_shape=[
            jax.ShapeDtypeStruct((4, N, CQ), jnp.float32),
            jax.ShapeDtypeStruct((4, N, CQ), jnp.float32),
            jax.ShapeDtypeStruct((2, C), jnp.float32),
        ],
        scratch_shapes=[pltpu.VMEM((2, C), jnp.float32)],
    )(feat_all, xyz, w1t, b1r, wpt, bpr)


def _k2_body(pw_hbm, y_hbm, ids_hbm, z_hbm, out_hbm,
             idv, pwv, yv, comb, tab):
    c = lax.axis_index("c")
    s = lax.axis_index("s")
    nr = S_PAD // NS
    rows0 = s * nr
    base0 = s * (N // NS)

    for qi in range(2):
        q = c * 2 + qi
        pltpu.sync_copy(z_hbm, tab.at[pl.ds(rows0, nr)])
        plsc.subcore_barrier()

        def chunk(j, carry):
            base = base0 + j * K2K
            pltpu.sync_copy(ids_hbm.at[pl.ds(base, K2K)], idv)
            pltpu.sync_copy(pw_hbm.at[q, pl.ds(base, K2K), :], pwv)
            pltpu.sync_copy(y_hbm.at[q, pl.ds(base, K2K), :], yv)

            def row(i, carry2):
                for g in range(CQ // 16):
                    sl = pl.ds(g * 16, 16)
                    pwr = pwv[i, sl]
                    yr = yv[i, sl]
                    comb[i, pl.ds(g * 16, 16)] = pwr
                    comb[i, pl.ds(CQ + g * 16, 16)] = pwr * yr
                    comb[i, pl.ds(2 * CQ + g * 16, 16)] = pwr * pwr
                return carry2

            lax.fori_loop(0, K2K, row, 0)
            pltpu.sync_copy(comb, tab.at[idv], add=True)
            return carry

        lax.fori_loop(0, (N // NS) // K2K, chunk, 0)
        plsc.subcore_barrier()
        pltpu.sync_copy(tab.at[pl.ds(rows0, nr)],
                        out_hbm.at[q, pl.ds(rows0, nr), :])
        plsc.subcore_barrier()


def _k2(pw_split, y_split, unq_inv, zeros):
    mesh = plsc.VectorSubcoreMesh(core_axis_name="c", subcore_axis_name="s",
                                  num_cores=NC, num_subcores=NS)
    return pl.kernel(
        _k2_body,
        out_type=jax.ShapeDtypeStruct((4, S_PAD, 3 * CQ), jnp.float32),
        mesh=mesh,
        scratch_types=[
            pltpu.VMEM((K2K,), jnp.int32),
            pltpu.VMEM((K2K, CQ), jnp.float32),
            pltpu.VMEM((K2K, CQ), jnp.float32),
            pltpu.VMEM((K2K, 3 * CQ), jnp.float32),
            pltpu.VMEM_SHARED((S_PAD, 3 * CQ), jnp.float32),
        ],
    )(pw_split, y_split, unq_inv, zeros)


def _k3_body(tabs_ref, stats_ref, gs_ref, r_ref, consts_ref, acc_ref):
    j = pl.program_id(0)
    mu1 = stats_ref[0:1, :] * (1.0 / N)
    var1 = stats_ref[1:2, :] * (1.0 / N) - mu1 * mu1
    a1 = gs_ref[0:1, :] * lax.rsqrt(var1 + EPS_BN)
    c1 = gs_ref[1:2, :] - a1 * mu1
    spw = jnp.concatenate([tabs_ref[q][:, 0:CQ] for q in range(4)], axis=1)
    spwy = jnp.concatenate([tabs_ref[q][:, CQ:2 * CQ] for q in range(4)],
                           axis=1)
    spw2 = jnp.concatenate([tabs_ref[q][:, 2 * CQ:3 * CQ] for q in range(4)],
                           axis=1)
    feat_add = a1 * spwy + c1 * spw
    r = -feat_add / (spw + 1e-4)
    r_ref[...] = r

    @pl.when(j == 0)
    def _():
        acc_ref[...] = jnp.zeros_like(acc_ref)

    acc_ref[0:1, :] = acc_ref[0:1, :] + jnp.sum(r * spw, axis=0, keepdims=True)
    acc_ref[1:2, :] = acc_ref[1:2, :] + jnp.sum(r * r * spw2, axis=0,
                                                keepdims=True)

    @pl.when(j == pl.num_programs(0) - 1)
    def _():
        mu2 = acc_ref[0:1, :] * (1.0 / N)
        var2 = acc_ref[1:2, :] * (1.0 / N) - mu2 * mu2
        a2 = gs_ref[2:3, :] * lax.rsqrt(var2 + EPS_BN)
        c2 = gs_ref[3:4, :] - a2 * mu2
        consts_ref[...] = jnp.concatenate([a1, c1, a2, c2], axis=0)


def _k3(tabs, stats, gs):
    nb = S_PAD // K3B
    return pl.pallas_call(
        _k3_body,
        grid=(nb,),
        in_specs=[
            pl.BlockSpec((4, K3B, 3 * CQ), lambda j: (0, j, 0)),
            pl.BlockSpec((2, C), lambda j: (0, 0)),
            pl.BlockSpec((4, C), lambda j: (0, 0)),
        ],
        out_specs=[
            pl.BlockSpec((K3B, C), lambda j: (j, 0)),
            pl.BlockSpec((4, C), lambda j: (0, 0)),
        ],
        out_shape=[
            jax.ShapeDtypeStruct((S_PAD, C), jnp.float32),
            jax.ShapeDtypeStruct((4, C), jnp.float32),
        ],
        scratch_shapes=[pltpu.VMEM((2, C), jnp.float32)],
    )(tabs, stats, gs)


def _k4_body(pw_hbm, y_hbm, ids_hbm, r_hbm, consts_hbm, out_hbm,
             idv, rgv, pv, yv, ov, cv, sem):
    c = lax.axis_index("c")
    s = lax.axis_index("s")
    w = s * NC + c
    rows_w = N // (NC * NS)
    base0 = w * rows_w
    pltpu.sync_copy(consts_hbm, cv)
    a1 = [cv[0, pl.ds(g * 16, 16)] for g in range(C // 16)]
    c1 = [cv[1, pl.ds(g * 16, 16)] for g in range(C // 16)]
    a2 = [cv[2, pl.ds(g * 16, 16)] for g in range(C // 16)]
    c2 = [cv[3, pl.ds(g * 16, 16)] for g in range(C // 16)]

    def chunk(j, carry):
        base = base0 + j * K4K
        pltpu.sync_copy(ids_hbm.at[pl.ds(base, K4K)], idv)
        pltpu.async_copy(r_hbm.at[idv], rgv, sem).wait()
        for q in range(4):
            pltpu.sync_copy(pw_hbm.at[q, pl.ds(base, K4K), :], pv.at[q])
            pltpu.sync_copy(y_hbm.at[q, pl.ds(base, K4K), :], yv.at[q])

        def row(i, carry2):
            for g in range(C // 16):
                q, gq = divmod(g, 2)
                slq = pl.ds(gq * 16, 16)
                sl = pl.ds(g * 16, 16)
                t = a2[g] * (pv[q, i, slq] * rgv[i, sl]) + c2[g]
                t = jnp.maximum(t, 0.0)
                ov[i, sl] = t + a1[g] * yv[q, i, slq] + c1[g]
            return carry2

        lax.fori_loop(0, K4K, row, 0)
        pltpu.sync_copy(ov, out_hbm.at[pl.ds(base, K4K), :])
        return carry

    lax.fori_loop(0, rows_w // K4K, chunk, 0)


def _k4(pw_split, y_split, unq_inv, r, consts):
    mesh = plsc.VectorSubcoreMesh(core_axis_name="c", subcore_axis_name="s",
                                  num_cores=NC, num_subcores=NS)
    return pl.kernel(
        _k4_body,
        out_type=jax.ShapeDtypeStruct((N, C), jnp.float32),
        mesh=mesh,
        scratch_types=[
            pltpu.VMEM((K4K,), jnp.int32),
            pltpu.VMEM((K4K, C), jnp.float32),
            pltpu.VMEM((4, K4K, CQ), jnp.float32),
            pltpu.VMEM((4, K4K, CQ), jnp.float32),
            pltpu.VMEM((K4K, C), jnp.float32),
            pltpu.VMEM((4, C), jnp.float32),
            pltpu.SemaphoreType.DMA,
        ],
    )(pw_split, y_split, unq_inv, r, consts)


def kernel(points_xyz, feat_all, unq_inv, W1, b1, g1, be1, Wp, bp, g2, be2):
    w1t = W1.T
    wpt = Wp.T
    b1r = b1.reshape(1, C)
    bpr = bp.reshape(1, C)
    gs = jnp.stack([g1, be1, g2, be2], axis=0)
    y_split, pw_split, stats = _k1(feat_all, points_xyz, w1t, b1r, wpt, bpr)
    zeros = jnp.zeros((S_PAD // NS, 3 * CQ), jnp.float32)
    tabs = _k2(pw_split, y_split, unq_inv, zeros)
    r, consts = _k3(tabs, stats, gs)
    return _k4(pw_split, y_split, unq_inv, r, consts)


# speedup vs baseline: 1.1371x; 1.1371x over previous
import jax
import jax.numpy as jnp
from jax import lax
from jax.experimental import pallas as pl
from jax.experimental.pallas import tpu as pltpu
from jax.experimental.pallas import tpu_sc as plsc

N = 320000
C = 128
CQ = 32
NSEG = 10000
S_PAD = 10112
EPS_BN = 1e-3
NC = 2
NS = 16
K1B = 2000
K2K = 80
K4K = 80
K3B = 1264


def _k1_body(x_ref, xyz_ref, w1t_ref, b1_ref, wpt_ref, bp_ref,
             y_ref, pw_ref, stats_ref, acc_ref):
    i = pl.program_id(0)
    y = jnp.dot(x_ref[...], w1t_ref[...],
                preferred_element_type=jnp.float32) + b1_ref[...]
    p = jnp.floor(xyz_ref[...])
    wpt = wpt_ref[...]
    pw = (p[:, 0:1] * wpt[0:1, :] + p[:, 1:2] * wpt[1:2, :]
          + p[:, 2:3] * wpt[2:3, :] + bp_ref[...])
    for q in range(4):
        y_ref[q] = y[:, q * CQ:(q + 1) * CQ]
        pw_ref[q] = pw[:, q * CQ:(q + 1) * CQ]

    @pl.when(i == 0)
    def _():
        acc_ref[...] = jnp.zeros_like(acc_ref)

    acc_ref[0:1, :] = acc_ref[0:1, :] + jnp.sum(y, axis=0, keepdims=True)
    acc_ref[1:2, :] = acc_ref[1:2, :] + jnp.sum(y * y, axis=0, keepdims=True)

    @pl.when(i == pl.num_programs(0) - 1)
    def _():
        stats_ref[...] = acc_ref[...]


def _k1(feat_all, xyz, w1t, b1r, wpt, bpr):
    nb = N // K1B
    return pl.pallas_call(
        _k1_body,
        grid=(nb,),
        in_specs=[
            pl.BlockSpec((K1B, C), lambda i: (i, 0)),
            pl.BlockSpec((K1B, 3), lambda i: (i, 0)),
            pl.BlockSpec((C, C), lambda i: (0, 0)),
            pl.BlockSpec((1, C), lambda i: (0, 0)),
            pl.BlockSpec((3, C), lambda i: (0, 0)),
            pl.BlockSpec((1, C), lambda i: (0, 0)),
        ],
        out_specs=[
            pl.BlockSpec((4, K1B, CQ), lambda i: (0, i, 0)),
            pl.BlockSpec((4, K1B, CQ), lambda i: (0, i, 0)),
            pl.BlockSpec((2, C), lambda i: (0, 0)),
        ],
        out_shape=[
            jax.ShapeDtypeStruct((4, N, CQ), jnp.float32),
            jax.ShapeDtypeStruct((4, N, CQ), jnp.float32),
            jax.ShapeDtypeStruct((2, C), jnp.float32),
        ],
        scratch_shapes=[pltpu.VMEM((2, C), jnp.float32)],
    )(feat_all, xyz, w1t, b1r, wpt, bpr)


def _k2_body(pw_hbm, y_hbm, ids_hbm, z_hbm, out_hbm,
             idv0, idv1, pwv, yv, comb0, comb1, tab, sem):
    c = lax.axis_index("c")
    s = lax.axis_index("s")
    nr = S_PAD // NS
    rows0 = s * nr
    base0 = s * (N // NS)

    for qi in range(2):
        q = c * 2 + qi
        pltpu.sync_copy(z_hbm, tab.at[pl.ds(rows0, nr)])
        plsc.subcore_barrier()

        def chunk(j, carry):
            base = base0 + j * K2K
            d = [
                pltpu.async_copy(ids_hbm.at[pl.ds(base, K2K)], idv0, sem),
                pltpu.async_copy(pw_hbm.at[q, pl.ds(base, K2K), :], pwv, sem),
                pltpu.async_copy(y_hbm.at[q, pl.ds(base, K2K), :], yv, sem),
            ]
            for dd in d:
                dd.wait()

            def row(i, carry2):
                for g in range(CQ // 16):
                    sl = pl.ds(g * 16, 16)
                    pwr = pwv[i, sl]
                    yr = yv[i, sl]
                    comb0[i, pl.ds(g * 16, 16)] = pwr
                    comb0[i, pl.ds(CQ + g * 16, 16)] = pwr * yr
                    comb0[i, pl.ds(2 * CQ + g * 16, 16)] = pwr * pwr
                return carry2

            lax.fori_loop(0, K2K, row, 0)
            pltpu.sync_copy(comb0, tab.at[idv0], add=True)
            return carry

        lax.fori_loop(0, (N // NS) // K2K, chunk, 0)
        plsc.subcore_barrier()
        pltpu.sync_copy(tab.at[pl.ds(rows0, nr)],
                        out_hbm.at[q, pl.ds(rows0, nr), :])
        plsc.subcore_barrier()


def _k2(pw_split, y_split, unq_inv, zeros):
    mesh = plsc.VectorSubcoreMesh(core_axis_name="c", subcore_axis_name="s",
                                  num_cores=NC, num_subcores=NS)
    return pl.kernel(
        _k2_body,
        out_type=jax.ShapeDtypeStruct((4, S_PAD, 3 * CQ), jnp.float32),
        mesh=mesh,
        scratch_types=[
            pltpu.VMEM((K2K,), jnp.int32),
            pltpu.VMEM((K2K,), jnp.int32),
            pltpu.VMEM((K2K, CQ), jnp.float32),
            pltpu.VMEM((K2K, CQ), jnp.float32),
            pltpu.VMEM((K2K, 3 * CQ), jnp.float32),
            pltpu.VMEM((K2K, 3 * CQ), jnp.float32),
            pltpu.VMEM_SHARED((S_PAD, 3 * CQ), jnp.float32),
            pltpu.SemaphoreType.DMA,
        ],
    )(pw_split, y_split, unq_inv, zeros)


def _k3_body(tabs_ref, stats_ref, gs_ref, r_ref, consts_ref, acc_ref):
    j = pl.program_id(0)
    mu1 = stats_ref[0:1, :] * (1.0 / N)
    var1 = stats_ref[1:2, :] * (1.0 / N) - mu1 * mu1
    a1 = gs_ref[0:1, :] * lax.rsqrt(var1 + EPS_BN)
    c1 = gs_ref[1:2, :] - a1 * mu1
    spw = jnp.concatenate([tabs_ref[q][:, 0:CQ] for q in range(4)], axis=1)
    spwy = jnp.concatenate([tabs_ref[q][:, CQ:2 * CQ] for q in range(4)],
                           axis=1)
    spw2 = jnp.concatenate([tabs_ref[q][:, 2 * CQ:3 * CQ] for q in range(4)],
                           axis=1)
    feat_add = a1 * spwy + c1 * spw
    r = -feat_add / (spw + 1e-4)
    r_ref[...] = r

    @pl.when(j == 0)
    def _():
        acc_ref[...] = jnp.zeros_like(acc_ref)

    acc_ref[0:1, :] = acc_ref[0:1, :] + jnp.sum(r * spw, axis=0, keepdims=True)
    acc_ref[1:2, :] = acc_ref[1:2, :] + jnp.sum(r * r * spw2, axis=0,
                                                keepdims=True)

    @pl.when(j == pl.num_programs(0) - 1)
    def _():
        mu2 = acc_ref[0:1, :] * (1.0 / N)
        var2 = acc_ref[1:2, :] * (1.0 / N) - mu2 * mu2
        a2 = gs_ref[2:3, :] * lax.rsqrt(var2 + EPS_BN)
        c2 = gs_ref[3:4, :] - a2 * mu2
        consts_ref[...] = jnp.concatenate([a1, c1, a2, c2], axis=0)


def _k3(tabs, stats, gs):
    nb = S_PAD // K3B
    return pl.pallas_call(
        _k3_body,
        grid=(nb,),
        in_specs=[
            pl.BlockSpec((4, K3B, 3 * CQ), lambda j: (0, j, 0)),
            pl.BlockSpec((2, C), lambda j: (0, 0)),
            pl.BlockSpec((4, C), lambda j: (0, 0)),
        ],
        out_specs=[
            pl.BlockSpec((K3B, C), lambda j: (j, 0)),
            pl.BlockSpec((4, C), lambda j: (0, 0)),
        ],
        out_shape=[
            jax.ShapeDtypeStruct((S_PAD, C), jnp.float32),
            jax.ShapeDtypeStruct((4, C), jnp.float32),
        ],
        scratch_shapes=[pltpu.VMEM((2, C), jnp.float32)],
    )(tabs, stats, gs)


def _k4_body(pw_hbm, y_hbm, ids_hbm, r_hbm, consts_hbm, out_hbm,
             idv, rgv, pv, yv, ov, cv, sem, semg):
    c = lax.axis_index("c")
    s = lax.axis_index("s")
    w = s * NC + c
    rows_w = N // (NC * NS)
    base0 = w * rows_w
    pltpu.sync_copy(consts_hbm, cv)
    a1 = [cv[0, pl.ds(g * 16, 16)] for g in range(C // 16)]
    c1 = [cv[1, pl.ds(g * 16, 16)] for g in range(C // 16)]
    a2 = [cv[2, pl.ds(g * 16, 16)] for g in range(C // 16)]
    c2 = [cv[3, pl.ds(g * 16, 16)] for g in range(C // 16)]

    def chunk(j, carry):
        base = base0 + j * K4K
        pltpu.sync_copy(ids_hbm.at[pl.ds(base, K4K)], idv)
        pltpu.async_copy(r_hbm.at[idv], rgv, semg).wait()
        for q in range(4):
            pltpu.sync_copy(pw_hbm.at[q, pl.ds(base, K4K), :], pv.at[q])
            pltpu.sync_copy(y_hbm.at[q, pl.ds(base, K4K), :], yv.at[q])

        def row(i, carry2):
            for g in range(C // 16):
                q, gq = divmod(g, 2)
                slq = pl.ds(gq * 16, 16)
                sl = pl.ds(g * 16, 16)
                t = a2[g] * (pv[q, i, slq] * rgv[i, sl]) + c2[g]
                t = jnp.maximum(t, 0.0)
                ov[i, sl] = t + a1[g] * yv[q, i, slq] + c1[g]
            return carry2

        lax.fori_loop(0, K4K, row, 0)
        pltpu.sync_copy(ov, out_hbm.at[pl.ds(base, K4K), :])
        return carry

    lax.fori_loop(0, rows_w // K4K, chunk, 0)


def _k4(pw_split, y_split, unq_inv, r, consts):
    mesh = plsc.VectorSubcoreMesh(core_axis_name="c", subcore_axis_name="s",
                                  num_cores=NC, num_subcores=NS)
    return pl.kernel(
        _k4_body,
        out_type=jax.ShapeDtypeStruct((N, C), jnp.float32),
        mesh=mesh,
        scratch_types=[
            pltpu.VMEM((K4K,), jnp.int32),
            pltpu.VMEM((K4K, C), jnp.float32),
            pltpu.VMEM((4, K4K, CQ), jnp.float32),
            pltpu.VMEM((4, K4K, CQ), jnp.float32),
            pltpu.VMEM((K4K, C), jnp.float32),
            pltpu.VMEM((4, C), jnp.float32),
            pltpu.SemaphoreType.DMA,
            pltpu.SemaphoreType.DMA,
        ],
    )(pw_split, y_split, unq_inv, r, consts)


def kernel(points_xyz, feat_all, unq_inv, W1, b1, g1, be1, Wp, bp, g2, be2):
    w1t = W1.T
    wpt = Wp.T
    b1r = b1.reshape(1, C)
    bpr = bp.reshape(1, C)
    gs = jnp.stack([g1, be1, g2, be2], axis=0)
    y_split, pw_split, stats = _k1(feat_all, points_xyz, w1t, b1r, wpt, bpr)
    zeros = jnp.zeros((S_PAD // NS, 3 * CQ), jnp.float32)
    tabs = _k2(pw_split, y_split, unq_inv, zeros)
    r, consts = _k3(tabs, stats, gs)
    return _k4(pw_split, y_split, unq_inv, r, consts)


# speedup vs baseline: 1.5527x; 1.3655x over previous
import jax
import jax.numpy as jnp
from jax import lax
from jax.experimental import pallas as pl
from jax.experimental.pallas import tpu as pltpu
from jax.experimental.pallas import tpu_sc as plsc

N = 320000
C = 128
CQ = 32
NSEG = 10000
S_PAD = 10112
EPS_BN = 1e-3
NC = 2
NS = 16
K1B = 2000
K2K = 80
K4K = 80
K3B = 1264


def _k1_body(x_ref, xyz_ref, w1t_ref, b1_ref, wpt_ref, bp_ref,
             y_ref, pw_ref, stats_ref, acc_ref):
    i = pl.program_id(0)
    y = jnp.dot(x_ref[...], w1t_ref[...],
                preferred_element_type=jnp.float32) + b1_ref[...]
    p = jnp.floor(xyz_ref[...])
    wpt = wpt_ref[...]
    pw = (p[:, 0:1] * wpt[0:1, :] + p[:, 1:2] * wpt[1:2, :]
          + p[:, 2:3] * wpt[2:3, :] + bp_ref[...])
    for q in range(4):
        y_ref[q] = y[:, q * CQ:(q + 1) * CQ]
        pw_ref[q] = pw[:, q * CQ:(q + 1) * CQ]

    @pl.when(i == 0)
    def _():
        acc_ref[...] = jnp.zeros_like(acc_ref)

    acc_ref[0:1, :] = acc_ref[0:1, :] + jnp.sum(y, axis=0, keepdims=True)
    acc_ref[1:2, :] = acc_ref[1:2, :] + jnp.sum(y * y, axis=0, keepdims=True)

    @pl.when(i == pl.num_programs(0) - 1)
    def _():
        stats_ref[...] = acc_ref[...]


def _k1(feat_all, xyz, w1t, b1r, wpt, bpr):
    nb = N // K1B
    return pl.pallas_call(
        _k1_body,
        grid=(nb,),
        in_specs=[
            pl.BlockSpec((K1B, C), lambda i: (i, 0)),
            pl.BlockSpec((K1B, 3), lambda i: (i, 0)),
            pl.BlockSpec((C, C), lambda i: (0, 0)),
            pl.BlockSpec((1, C), lambda i: (0, 0)),
            pl.BlockSpec((3, C), lambda i: (0, 0)),
            pl.BlockSpec((1, C), lambda i: (0, 0)),
        ],
        out_specs=[
            pl.BlockSpec((4, K1B, CQ), lambda i: (0, i, 0)),
            pl.BlockSpec((4, K1B, CQ), lambda i: (0, i, 0)),
            pl.BlockSpec((2, C), lambda i: (0, 0)),
        ],
        out_shape=[
            jax.ShapeDtypeStruct((4, N, CQ), jnp.float32),
            jax.ShapeDtypeStruct((4, N, CQ), jnp.float32),
            jax.ShapeDtypeStruct((2, C), jnp.float32),
        ],
        scratch_shapes=[pltpu.VMEM((2, C), jnp.float32)],
    )(feat_all, xyz, w1t, b1r, wpt, bpr)


def _k2_body(pw_hbm, y_hbm, ids_hbm, z_hbm, out_hbm,
             idv0, idv1, pwv, yv, comb0, comb1, tab, sem):
    c = lax.axis_index("c")
    s = lax.axis_index("s")
    nr = S_PAD // NS
    rows0 = s * nr
    base0 = s * (N // NS)

    for qi in range(2):
        q = c * 2 + qi
        pltpu.sync_copy(z_hbm, tab.at[pl.ds(rows0, nr)])
        plsc.subcore_barrier()

        def chunk(j, carry):
            base = base0 + j * K2K
            d = [
                pltpu.async_copy(ids_hbm.at[pl.ds(base, K2K)], idv0, sem),
                pltpu.async_copy(pw_hbm.at[q, pl.ds(base, K2K), :], pwv, sem),
                pltpu.async_copy(y_hbm.at[q, pl.ds(base, K2K), :], yv, sem),
            ]
            for dd in d:
                dd.wait()

            def row(i, carry2):
                for g in range(CQ // 16):
                    sl = pl.ds(g * 16, 16)
                    pwr = pwv[i, sl]
                    yr = yv[i, sl]
                    comb0[i, pl.ds(g * 16, 16)] = pwr
                    comb0[i, pl.ds(CQ + g * 16, 16)] = pwr * yr
                    comb0[i, pl.ds(2 * CQ + g * 16, 16)] = pwr * pwr
                return carry2

            lax.fori_loop(0, K2K, row, 0)
            pltpu.sync_copy(comb0, tab.at[idv0], add=True)
            return carry

        lax.fori_loop(0, (N // NS) // K2K, chunk, 0)
        plsc.subcore_barrier()
        pltpu.sync_copy(tab.at[pl.ds(rows0, nr)],
                        out_hbm.at[q, pl.ds(rows0, nr), :])
        plsc.subcore_barrier()


def _k2(pw_split, y_split, unq_inv, zeros):
    mesh = plsc.VectorSubcoreMesh(core_axis_name="c", subcore_axis_name="s",
                                  num_cores=NC, num_subcores=NS)
    return pl.kernel(
        _k2_body,
        out_type=jax.ShapeDtypeStruct((4, S_PAD, 3 * CQ), jnp.float32),
        mesh=mesh,
        scratch_types=[
            pltpu.VMEM((K2K,), jnp.int32),
            pltpu.VMEM((K2K,), jnp.int32),
            pltpu.VMEM((K2K, CQ), jnp.float32),
            pltpu.VMEM((K2K, CQ), jnp.float32),
            pltpu.VMEM((K2K, 3 * CQ), jnp.float32),
            pltpu.VMEM((K2K, 3 * CQ), jnp.float32),
            pltpu.VMEM_SHARED((S_PAD, 3 * CQ), jnp.float32),
            pltpu.SemaphoreType.DMA,
        ],
    )(pw_split, y_split, unq_inv, zeros)


def _k3_body(tabs_ref, stats_ref, gs_ref, r_ref, consts_ref, acc_ref):
    j = pl.program_id(0)
    mu1 = stats_ref[0:1, :] * (1.0 / N)
    var1 = stats_ref[1:2, :] * (1.0 / N) - mu1 * mu1
    a1 = gs_ref[0:1, :] * lax.rsqrt(var1 + EPS_BN)
    c1 = gs_ref[1:2, :] - a1 * mu1
    spw = jnp.concatenate([tabs_ref[q][:, 0:CQ] for q in range(4)], axis=1)
    spwy = jnp.concatenate([tabs_ref[q][:, CQ:2 * CQ] for q in range(4)],
                           axis=1)
    spw2 = jnp.concatenate([tabs_ref[q][:, 2 * CQ:3 * CQ] for q in range(4)],
                           axis=1)
    feat_add = a1 * spwy + c1 * spw
    r = -feat_add / (spw + 1e-4)
    r_ref[...] = r

    @pl.when(j == 0)
    def _():
        acc_ref[...] = jnp.zeros_like(acc_ref)

    acc_ref[0:1, :] = acc_ref[0:1, :] + jnp.sum(r * spw, axis=0, keepdims=True)
    acc_ref[1:2, :] = acc_ref[1:2, :] + jnp.sum(r * r * spw2, axis=0,
                                                keepdims=True)

    @pl.when(j == pl.num_programs(0) - 1)
    def _():
        mu2 = acc_ref[0:1, :] * (1.0 / N)
        var2 = acc_ref[1:2, :] * (1.0 / N) - mu2 * mu2
        a2 = gs_ref[2:3, :] * lax.rsqrt(var2 + EPS_BN)
        c2 = gs_ref[3:4, :] - a2 * mu2
        consts_ref[...] = jnp.concatenate([a1, c1, a2, c2], axis=0)


def _k3(tabs, stats, gs):
    nb = S_PAD // K3B
    return pl.pallas_call(
        _k3_body,
        grid=(nb,),
        in_specs=[
            pl.BlockSpec((4, K3B, 3 * CQ), lambda j: (0, j, 0)),
            pl.BlockSpec((2, C), lambda j: (0, 0)),
            pl.BlockSpec((4, C), lambda j: (0, 0)),
        ],
        out_specs=[
            pl.BlockSpec((K3B, C), lambda j: (j, 0)),
            pl.BlockSpec((4, C), lambda j: (0, 0)),
        ],
        out_shape=[
            jax.ShapeDtypeStruct((S_PAD, C), jnp.float32),
            jax.ShapeDtypeStruct((4, C), jnp.float32),
        ],
        scratch_shapes=[pltpu.VMEM((2, C), jnp.float32)],
    )(tabs, stats, gs)


def _k4_body(pw_hbm, y_hbm, ids_hbm, r_hbm, consts_hbm, out_hbm,
             idv, rgv, pv, yv, ov, cv, sem, semg):
    c = lax.axis_index("c")
    s = lax.axis_index("s")
    w = s * NC + c
    rows_w = N // (NC * NS)
    base0 = w * rows_w
    pltpu.sync_copy(consts_hbm, cv)
    a1 = [cv[0, pl.ds(g * 16, 16)] for g in range(C // 16)]
    c1 = [cv[1, pl.ds(g * 16, 16)] for g in range(C // 16)]
    a2 = [cv[2, pl.ds(g * 16, 16)] for g in range(C // 16)]
    c2 = [cv[3, pl.ds(g * 16, 16)] for g in range(C // 16)]

    def chunk(j, carry):
        base = base0 + j * K4K
        pltpu.sync_copy(ids_hbm.at[pl.ds(base, K4K)], idv)
        d = [pltpu.async_copy(r_hbm.at[idv], rgv, semg)]
        for q in range(4):
            d.append(pltpu.async_copy(pw_hbm.at[q, pl.ds(base, K4K), :],
                                      pv.at[q], sem))
            d.append(pltpu.async_copy(y_hbm.at[q, pl.ds(base, K4K), :],
                                      yv.at[q], sem))
        for dd in d:
            dd.wait()

        def row(i, carry2):
            for g in range(C // 16):
                q, gq = divmod(g, 2)
                slq = pl.ds(gq * 16, 16)
                sl = pl.ds(g * 16, 16)
                t = a2[g] * (pv[q, i, slq] * rgv[i, sl]) + c2[g]
                t = jnp.maximum(t, 0.0)
                ov[i, sl] = t + a1[g] * yv[q, i, slq] + c1[g]
            return carry2

        lax.fori_loop(0, K4K, row, 0)
        pltpu.sync_copy(ov, out_hbm.at[pl.ds(base, K4K), :])
        return carry

    lax.fori_loop(0, rows_w // K4K, chunk, 0)


def _k4(pw_split, y_split, unq_inv, r, consts):
    mesh = plsc.VectorSubcoreMesh(core_axis_name="c", subcore_axis_name="s",
                                  num_cores=NC, num_subcores=NS)
    return pl.kernel(
        _k4_body,
        out_type=jax.ShapeDtypeStruct((N, C), jnp.float32),
        mesh=mesh,
        scratch_types=[
            pltpu.VMEM((K4K,), jnp.int32),
            pltpu.VMEM((K4K, C), jnp.float32),
            pltpu.VMEM((4, K4K, CQ), jnp.float32),
            pltpu.VMEM((4, K4K, CQ), jnp.float32),
            pltpu.VMEM((K4K, C), jnp.float32),
            pltpu.VMEM((4, C), jnp.float32),
            pltpu.SemaphoreType.DMA,
            pltpu.SemaphoreType.DMA,
        ],
    )(pw_split, y_split, unq_inv, r, consts)


def kernel(points_xyz, feat_all, unq_inv, W1, b1, g1, be1, Wp, bp, g2, be2):
    w1t = W1.T
    wpt = Wp.T
    b1r = b1.reshape(1, C)
    bpr = bp.reshape(1, C)
    gs = jnp.stack([g1, be1, g2, be2], axis=0)
    y_split, pw_split, stats = _k1(feat_all, points_xyz, w1t, b1r, wpt, bpr)
    zeros = jnp.zeros((S_PAD // NS, 3 * CQ), jnp.float32)
    tabs = _k2(pw_split, y_split, unq_inv, zeros)
    r, consts = _k3(tabs, stats, gs)
    return _k4(pw_split, y_split, unq_inv, r, consts)


# speedup vs baseline: 1.6039x; 1.0329x over previous
import jax
import jax.numpy as jnp
from jax import lax
from jax.experimental import pallas as pl
from jax.experimental.pallas import tpu as pltpu
from jax.experimental.pallas import tpu_sc as plsc

N = 320000
C = 128
CQ = 32
NSEG = 10000
S_PAD = 10112
EPS_BN = 1e-3
NC = 2
NS = 16
K1B = 2000
K2K = 80
K4K = 80
K3B = 1264


def _k1_body(x_ref, xyz_ref, w1t_ref, b1_ref, wpt_ref, bp_ref,
             y_ref, pw_ref, stats_ref, acc_ref):
    i = pl.program_id(0)
    y = jnp.dot(x_ref[...], w1t_ref[...],
                preferred_element_type=jnp.float32) + b1_ref[...]
    p = jnp.floor(xyz_ref[...])
    wpt = wpt_ref[...]
    pw = (p[:, 0:1] * wpt[0:1, :] + p[:, 1:2] * wpt[1:2, :]
          + p[:, 2:3] * wpt[2:3, :] + bp_ref[...])
    for q in range(4):
        y_ref[q] = y[:, q * CQ:(q + 1) * CQ]
        pw_ref[q] = pw[:, q * CQ:(q + 1) * CQ]

    @pl.when(i == 0)
    def _():
        acc_ref[...] = jnp.zeros_like(acc_ref)

    acc_ref[0:1, :] = acc_ref[0:1, :] + jnp.sum(y, axis=0, keepdims=True)
    acc_ref[1:2, :] = acc_ref[1:2, :] + jnp.sum(y * y, axis=0, keepdims=True)

    @pl.when(i == pl.num_programs(0) - 1)
    def _():
        stats_ref[...] = acc_ref[...]


def _k1(feat_all, xyz, w1t, b1r, wpt, bpr):
    nb = N // K1B
    return pl.pallas_call(
        _k1_body,
        grid=(nb,),
        in_specs=[
            pl.BlockSpec((K1B, C), lambda i: (i, 0)),
            pl.BlockSpec((K1B, 3), lambda i: (i, 0)),
            pl.BlockSpec((C, C), lambda i: (0, 0)),
            pl.BlockSpec((1, C), lambda i: (0, 0)),
            pl.BlockSpec((3, C), lambda i: (0, 0)),
            pl.BlockSpec((1, C), lambda i: (0, 0)),
        ],
        out_specs=[
            pl.BlockSpec((4, K1B, CQ), lambda i: (0, i, 0)),
            pl.BlockSpec((4, K1B, CQ), lambda i: (0, i, 0)),
            pl.BlockSpec((2, C), lambda i: (0, 0)),
        ],
        out_shape=[
            jax.ShapeDtypeStruct((4, N, CQ), jnp.float32),
            jax.ShapeDtypeStruct((4, N, CQ), jnp.float32),
            jax.ShapeDtypeStruct((2, C), jnp.float32),
        ],
        scratch_shapes=[pltpu.VMEM((2, C), jnp.float32)],
    )(feat_all, xyz, w1t, b1r, wpt, bpr)


def _k2_body(pw_hbm, y_hbm, ids_hbm, z_hbm, out_hbm,
             idv0, idv1, pwv, yv, comb0, comb1, tab, sem):
    c = lax.axis_index("c")
    s = lax.axis_index("s")
    nr = S_PAD // NS
    rows0 = s * nr
    base0 = s * (N // NS)

    for qi in range(2):
        q = c * 2 + qi
        pltpu.sync_copy(z_hbm, tab.at[pl.ds(rows0, nr)])
        plsc.subcore_barrier()

        def chunk(j, carry):
            base = base0 + j * K2K
            d = [
                pltpu.async_copy(ids_hbm.at[pl.ds(base, K2K)], idv0, sem),
                pltpu.async_copy(pw_hbm.at[q, pl.ds(base, K2K), :], pwv, sem),
                pltpu.async_copy(y_hbm.at[q, pl.ds(base, K2K), :], yv, sem),
            ]
            for dd in d:
                dd.wait()

            def row(i, carry2):
                for g in range(CQ // 16):
                    sl = pl.ds(g * 16, 16)
                    pwr = pwv[i, sl]
                    yr = yv[i, sl]
                    comb0[i, pl.ds(g * 16, 16)] = pwr
                    comb0[i, pl.ds(CQ + g * 16, 16)] = pwr * yr
                    comb0[i, pl.ds(2 * CQ + g * 16, 16)] = pwr * pwr
                return carry2

            lax.fori_loop(0, K2K, row, 0)
            pltpu.sync_copy(comb0, tab.at[idv0], add=True)
            return carry

        lax.fori_loop(0, (N // NS) // K2K, chunk, 0)
        plsc.subcore_barrier()
        pltpu.sync_copy(tab.at[pl.ds(rows0, nr)],
                        out_hbm.at[q, pl.ds(rows0, nr), :])
        plsc.subcore_barrier()


def _k2(pw_split, y_split, unq_inv, zeros):
    mesh = plsc.VectorSubcoreMesh(core_axis_name="c", subcore_axis_name="s",
                                  num_cores=NC, num_subcores=NS)
    return pl.kernel(
        _k2_body,
        out_type=jax.ShapeDtypeStruct((4, S_PAD, 3 * CQ), jnp.float32),
        mesh=mesh,
        scratch_types=[
            pltpu.VMEM((K2K,), jnp.int32),
            pltpu.VMEM((K2K,), jnp.int32),
            pltpu.VMEM((K2K, CQ), jnp.float32),
            pltpu.VMEM((K2K, CQ), jnp.float32),
            pltpu.VMEM((K2K, 3 * CQ), jnp.float32),
            pltpu.VMEM((K2K, 3 * CQ), jnp.float32),
            pltpu.VMEM_SHARED((S_PAD, 3 * CQ), jnp.float32),
            pltpu.SemaphoreType.DMA,
        ],
    )(pw_split, y_split, unq_inv, zeros)


def _k3_body(tabs_ref, stats_ref, gs_ref, r_ref, consts_ref, acc_ref):
    j = pl.program_id(0)
    mu1 = stats_ref[0:1, :] * (1.0 / N)
    var1 = stats_ref[1:2, :] * (1.0 / N) - mu1 * mu1
    a1 = gs_ref[0:1, :] * lax.rsqrt(var1 + EPS_BN)
    c1 = gs_ref[1:2, :] - a1 * mu1
    spw = jnp.concatenate([tabs_ref[q][:, 0:CQ] for q in range(4)], axis=1)
    spwy = jnp.concatenate([tabs_ref[q][:, CQ:2 * CQ] for q in range(4)],
                           axis=1)
    spw2 = jnp.concatenate([tabs_ref[q][:, 2 * CQ:3 * CQ] for q in range(4)],
                           axis=1)
    feat_add = a1 * spwy + c1 * spw
    r = -feat_add / (spw + 1e-4)
    r_ref[...] = r

    @pl.when(j == 0)
    def _():
        acc_ref[...] = jnp.zeros_like(acc_ref)

    acc_ref[0:1, :] = acc_ref[0:1, :] + jnp.sum(r * spw, axis=0, keepdims=True)
    acc_ref[1:2, :] = acc_ref[1:2, :] + jnp.sum(r * r * spw2, axis=0,
                                                keepdims=True)

    @pl.when(j == pl.num_programs(0) - 1)
    def _():
        mu2 = acc_ref[0:1, :] * (1.0 / N)
        var2 = acc_ref[1:2, :] * (1.0 / N) - mu2 * mu2
        a2 = gs_ref[2:3, :] * lax.rsqrt(var2 + EPS_BN)
        c2 = gs_ref[3:4, :] - a2 * mu2
        consts_ref[...] = jnp.concatenate([a1, c1, a2, c2], axis=0)


def _k3(tabs, stats, gs):
    nb = S_PAD // K3B
    return pl.pallas_call(
        _k3_body,
        grid=(nb,),
        in_specs=[
            pl.BlockSpec((4, K3B, 3 * CQ), lambda j: (0, j, 0)),
            pl.BlockSpec((2, C), lambda j: (0, 0)),
            pl.BlockSpec((4, C), lambda j: (0, 0)),
        ],
        out_specs=[
            pl.BlockSpec((K3B, C), lambda j: (j, 0)),
            pl.BlockSpec((4, C), lambda j: (0, 0)),
        ],
        out_shape=[
            jax.ShapeDtypeStruct((S_PAD, C), jnp.float32),
            jax.ShapeDtypeStruct((4, C), jnp.float32),
        ],
        scratch_shapes=[pltpu.VMEM((2, C), jnp.float32)],
    )(tabs, stats, gs)


def _k4_body(pw_hbm, y_hbm, ids_hbm, r_hbm, consts_hbm, out_hbm,
             idv, rgv, pv, yv, ov, cv, sem, semg, semi, semo):
    c = lax.axis_index("c")
    s = lax.axis_index("s")
    w = s * NC + c
    rows_w = N // (NC * NS)
    base0 = w * rows_w
    nch = rows_w // K4K
    pltpu.sync_copy(consts_hbm, cv)
    a1 = [cv[0, pl.ds(g * 16, 16)] for g in range(C // 16)]
    c1 = [cv[1, pl.ds(g * 16, 16)] for g in range(C // 16)]
    a2 = [cv[2, pl.ds(g * 16, 16)] for g in range(C // 16)]
    c2 = [cv[3, pl.ds(g * 16, 16)] for g in range(C // 16)]

    pltpu.async_copy(ids_hbm.at[pl.ds(base0, K4K)], idv.at[0], semi)
    pltpu.async_copy(ov.at[1], out_hbm.at[pl.ds(base0, K4K), :], semo)

    def chunk(j, b):
        base = base0 + j * K4K
        nxt = base0 + jnp.minimum(j + 1, nch - 1) * K4K
        pltpu.make_async_copy(ids_hbm.at[pl.ds(base, K4K)], idv.at[b],
                              semi).wait()
        pltpu.async_copy(ids_hbm.at[pl.ds(nxt, K4K)], idv.at[1 - b], semi)
        d = [pltpu.async_copy(r_hbm.at[idv.at[b]], rgv, semg)]
        for q in range(4):
            d.append(pltpu.async_copy(pw_hbm.at[q, pl.ds(base, K4K), :],
                                      pv.at[q], sem))
            d.append(pltpu.async_copy(y_hbm.at[q, pl.ds(base, K4K), :],
                                      yv.at[q], sem))
        pltpu.make_async_copy(ov.at[1 - b], out_hbm.at[pl.ds(base, K4K), :],
                              semo).wait()
        for dd in d:
            dd.wait()

        @plsc.parallel_loop(0, K4K, unroll=2)
        def row(i):
            for g in range(C // 16):
                q, gq = divmod(g, 2)
                slq = pl.ds(gq * 16, 16)
                sl = pl.ds(g * 16, 16)
                t = a2[g] * (pv[q, i, slq] * rgv[i, sl]) + c2[g]
                t = jnp.maximum(t, 0.0)
                ov[b, i, sl] = t + a1[g] * yv[q, i, slq] + c1[g]

        pltpu.async_copy(ov.at[b], out_hbm.at[pl.ds(base, K4K), :], semo)

    def pair(pj, carry):
        chunk(2 * pj, 0)
        chunk(2 * pj + 1, 1)
        return carry

    lax.fori_loop(0, nch // 2, pair, 0)
    chunk(nch - 1, 0)
    lastb = base0 + (nch - 1) * K4K
    pltpu.make_async_copy(ids_hbm.at[pl.ds(lastb, K4K)], idv.at[1],
                          semi).wait()
    pltpu.make_async_copy(ov.at[0], out_hbm.at[pl.ds(lastb, K4K), :],
                          semo).wait()


def _k4(pw_split, y_split, unq_inv, r, consts):
    mesh = plsc.VectorSubcoreMesh(core_axis_name="c", subcore_axis_name="s",
                                  num_cores=NC, num_subcores=NS)
    return pl.kernel(
        _k4_body,
        out_type=jax.ShapeDtypeStruct((N, C), jnp.float32),
        mesh=mesh,
        scratch_types=[
            pltpu.VMEM((2, K4K), jnp.int32),
            pltpu.VMEM((K4K, C), jnp.float32),
            pltpu.VMEM((4, K4K, CQ), jnp.float32),
            pltpu.VMEM((4, K4K, CQ), jnp.float32),
            pltpu.VMEM((2, K4K, C), jnp.float32),
            pltpu.VMEM((4, C), jnp.float32),
            pltpu.SemaphoreType.DMA,
            pltpu.SemaphoreType.DMA,
            pltpu.SemaphoreType.DMA,
            pltpu.SemaphoreType.DMA,
        ],
    )(pw_split, y_split, unq_inv, r, consts)


def kernel(points_xyz, feat_all, unq_inv, W1, b1, g1, be1, Wp, bp, g2, be2):
    w1t = W1.T
    wpt = Wp.T
    b1r = b1.reshape(1, C)
    bpr = bp.reshape(1, C)
    gs = jnp.stack([g1, be1, g2, be2], axis=0)
    y_split, pw_split, stats = _k1(feat_all, points_xyz, w1t, b1r, wpt, bpr)
    zeros = jnp.zeros((S_PAD // NS, 3 * CQ), jnp.float32)
    tabs = _k2(pw_split, y_split, unq_inv, zeros)
    r, consts = _k3(tabs, stats, gs)
    return _k4(pw_split, y_split, unq_inv, r, consts)
